# Initial kernel scaffold; baseline (speedup 1.0000x reference)
import jax
import jax.numpy as jnp
from jax.experimental import pallas as pl


def kernel(x, edge_index, edge_attr, Wq, bq, Wk, bk, Wv, bv, Wo, bo, Weq, beq, Wev, bev, gamma, beta):
    raise NotImplementedError("write your pallas kernel here")


# speedup vs baseline: 1.3041x; 1.3041x over previous
import functools

import jax
import jax.numpy as jnp
import numpy as np
from jax import lax
from jax.experimental import pallas as pl
from jax.experimental.pallas import tpu as pltpu
from jax.experimental.pallas import tpu_sc as plsc

N = 10000
E = 320000
D = 128
ED = 16
H = 8
HD = 16
INV_SCALE = 1.0 / float(np.sqrt(HD))

NC = 2
NS = 16
NW = NC * NS
EPT = E // NW
CH = 16
NG = CH // 16
NCHUNK = EPT // CH
N_PAD = 10240
NP4 = N_PAD // 4
ACC_ROWS = N_PAD + NP4
TROWS = ACC_ROWS // NS
ZR = 80


def _pre_body(x_ref, wqv_ref, bqv_ref, wk_ref, bk_ref, m_ref, qv_ref, kc_ref):
    xb = x_ref[...]
    qv_ref[...] = xb @ wqv_ref[...] + bqv_ref[...]
    k = xb @ wk_ref[...] + bk_ref[...]
    kc_ref[...] = jnp.concatenate([k, k @ m_ref[...]], axis=1)


def _tc_pre(x, wqv, bqv, wk, bk, m):
    bn = 1000
    grid = (N // bn,)
    return pl.pallas_call(
        _pre_body,
        grid=grid,
        in_specs=[
            pl.BlockSpec((bn, D), lambda i: (i, 0)),
            pl.BlockSpec((D, 2 * D), lambda i: (0, 0)),
            pl.BlockSpec((1, 2 * D), lambda i: (0, 0)),
            pl.BlockSpec((D, D), lambda i: (0, 0)),
            pl.BlockSpec((1, D), lambda i: (0, 0)),
            pl.BlockSpec((D, D), lambda i: (0, 0)),
        ],
        out_specs=[
            pl.BlockSpec((bn, 2 * D), lambda i: (i, 0)),
            pl.BlockSpec((bn, 2 * D), lambda i: (i, 0)),
        ],
        out_shape=[
            jax.ShapeDtypeStruct((N, 2 * D), jnp.float32),
            jax.ShapeDtypeStruct((N, 2 * D), jnp.float32),
        ],
    )(x, wqv, bqv, wk, bk, m)


def _sc_edge_kernel(qv_hbm, kc_hbm, src_hbm, dst_hbm, dst4_hbm, ea_hbm,
                    out_hbm,
                    src_idx, dst_idx, dst4_idx, qv_buf, kc_buf, ea_buf,
                    wv_buf, wp_buf, zero_buf, acc, sem1, sem2):
    cid = lax.axis_index("c")
    sub = lax.axis_index("s")
    wid = cid * NS + sub

    zeros16 = jnp.zeros((16,), jnp.float32)
    iota16 = lax.iota(jnp.int32, 16)
    rot_idx = [(iota16 + k) & 15 for k in (8, 4, 2, 1)]

    def zrow(r, carry):
        for j in range(D // 16):
            zero_buf[r, pl.ds(j * 16, 16)] = zeros16
        return carry
    lax.fori_loop(0, ZR, zrow, 0)
    t0 = sub * TROWS
    for i in range(TROWS // ZR):
        pltpu.sync_copy(zero_buf, acc.at[pl.ds(t0 + i * ZR, ZR)])
    plsc.subcore_barrier()

    def group_body(g, carry):
        dvec = dst_idx[pl.ds(g * 16, 16)]
        for j in range(16):
            e = g * 16 + j
            d = dvec[j]
            ea_v = ea_buf[e, :]
            esum = jnp.zeros((16,), jnp.float32)
            for h in range(H):
                q = qv_buf[e, pl.ds(h * HD, 16)]
                k = kc_buf[e, pl.ds(h * HD, 16)]
                c = kc_buf[e, pl.ds(D + h * HD, 16)]
                p = q * k + ea_v * c
                for ix in rot_idx:
                    p = p + p[ix]
                sv = p * INV_SCALE
                sv = jnp.minimum(jnp.maximum(sv, -5.0), 5.0)
                esum = esum + jnp.exp(sv)
            attn = esum * 0.125
            for h in range(H):
                v = qv_buf[e, pl.ds(D + h * HD, 16)]
                wv_buf[e, pl.ds(h * HD, 16)] = v * attn
            ea_a = ea_v * attn
            at_l0 = jnp.where(iota16 == 0, attn, 0.0)
            for blk in range(4):
                sel = (d & 3) == blk
                wp_buf[e, pl.ds(blk * 32, 16)] = jnp.where(sel, ea_a, 0.0)
                wp_buf[e, pl.ds(blk * 32 + 16, 16)] = jnp.where(sel, at_l0, 0.0)
        return carry

    def chunk(i, carry):
        base = wid * EPT + i * CH
        pltpu.sync_copy(src_hbm.at[pl.ds(base, CH)], src_idx)
        pltpu.sync_copy(dst_hbm.at[pl.ds(base, CH)], dst_idx)
        pltpu.sync_copy(dst4_hbm.at[pl.ds(base, CH)], dst4_idx)
        pltpu.sync_copy(ea_hbm.at[pl.ds(base, CH)], ea_buf)
        cp1 = pltpu.async_copy(qv_hbm.at[src_idx], qv_buf, sem1)
        cp2 = pltpu.async_copy(kc_hbm.at[dst_idx], kc_buf, sem2)
        cp1.wait()
        cp2.wait()
        lax.fori_loop(0, NG, group_body, 0)
        pltpu.sync_copy(wv_buf, acc.at[dst_idx], add=True)
        pltpu.sync_copy(wp_buf, acc.at[dst4_idx], add=True)
        return carry

    lax.fori_loop(0, NCHUNK, chunk, 0)

    plsc.subcore_barrier()
    pltpu.sync_copy(acc.at[pl.ds(t0, TROWS)],
                    out_hbm.at[cid, pl.ds(t0, TROWS)])


def _sc_edge(qv, kc, src, dst, dst4, ea):
    mesh = plsc.VectorSubcoreMesh(core_axis_name="c", subcore_axis_name="s")
    f = functools.partial(
        pl.kernel,
        out_type=jax.ShapeDtypeStruct((NC, ACC_ROWS, D), jnp.float32),
        mesh=mesh,
        scratch_types=[
            pltpu.VMEM((CH,), jnp.int32),
            pltpu.VMEM((CH,), jnp.int32),
            pltpu.VMEM((CH,), jnp.int32),
            pltpu.VMEM((CH, 2 * D), jnp.float32),
            pltpu.VMEM((CH, 2 * D), jnp.float32),
            pltpu.VMEM((CH, ED), jnp.float32),
            pltpu.VMEM((CH, D), jnp.float32),
            pltpu.VMEM((CH, D), jnp.float32),
            pltpu.VMEM((ZR, D), jnp.float32),
            pltpu.VMEM_SHARED((ACC_ROWS, D), jnp.float32),
            pltpu.SemaphoreType.DMA,
            pltpu.SemaphoreType.DMA,
        ],
    )(_sc_edge_kernel)
    return f(qv, kc, src, dst, dst4, ea)


def _post_body(v0_ref, v1_ref, s0_ref, s1_ref, x_ref, wev_ref, bev_ref,
               wo_ref, bo_ref, g_ref, b_ref, y_ref):
    a = v0_ref[...] + v1_ref[...]
    s = s0_ref[...] + s1_ref[...]
    bm = s[:, :ED]
    nrm = s[:, ED:ED + 1]
    acc = a + bm @ wev_ref[...] + nrm * bev_ref[...]
    out = acc / (nrm + 1e-8)
    out = out @ wo_ref[...] + bo_ref[...] + x_ref[...]
    mu = jnp.mean(out, axis=1, keepdims=True)
    var = jnp.mean((out - mu) ** 2, axis=1, keepdims=True)
    y_ref[...] = (out - mu) * lax.rsqrt(var + 1e-5) * g_ref[...] + b_ref[...]


def _tc_post(v0, v1, s0, s1, x, wev, bev, wo, bo, gamma, beta):
    bn = 1000
    grid = (N // bn,)
    return pl.pallas_call(
        _post_body,
        grid=grid,
        in_specs=[
            pl.BlockSpec((bn, D), lambda i: (i, 0)),
            pl.BlockSpec((bn, D), lambda i: (i, 0)),
            pl.BlockSpec((bn, 32), lambda i: (i, 0)),
            pl.BlockSpec((bn, 32), lambda i: (i, 0)),
            pl.BlockSpec((bn, D), lambda i: (i, 0)),
            pl.BlockSpec((ED, D), lambda i: (0, 0)),
            pl.BlockSpec((1, D), lambda i: (0, 0)),
            pl.BlockSpec((D, D), lambda i: (0, 0)),
            pl.BlockSpec((1, D), lambda i: (0, 0)),
            pl.BlockSpec((1, D), lambda i: (0, 0)),
            pl.BlockSpec((1, D), lambda i: (0, 0)),
        ],
        out_specs=pl.BlockSpec((bn, D), lambda i: (i, 0)),
        out_shape=jax.ShapeDtypeStruct((N, D), jnp.float32),
    )(v0, v1, s0, s1, x, wev, bev, wo, bo, gamma, beta)


def kernel(x, edge_index, edge_attr, Wq, bq, Wk, bk, Wv, bv, Wo, bo,
           Weq, beq, Wev, bev, gamma, beta):
    wqv = jnp.concatenate([Wq, Wv], axis=1)
    bqv = jnp.concatenate([bq + beq, bv])[None, :]
    blocks = [Weq[:, h * HD:(h + 1) * HD].T for h in range(H)]
    m = jax.scipy.linalg.block_diag(*blocks)

    src = edge_index[0]
    dst = edge_index[1]
    dst4 = jax.lax.shift_right_logical(dst, 2) + N_PAD

    qv, kc = _tc_pre(x, wqv, bqv, Wk, bk[None, :], m)
    acc = _sc_edge(qv, kc, src, dst, dst4, edge_attr)
    acc_v = acc[:, :N_PAD]
    sp = acc[:, N_PAD:].reshape(NC, N_PAD, 32)
    y = _tc_post(acc_v[0, :N], acc_v[1, :N], sp[0, :N], sp[1, :N], x,
                 Wev, bev[None, :], Wo, bo[None, :], gamma[None, :],
                 beta[None, :])
    return y


# speedup vs baseline: 2.9002x; 2.2239x over previous
import functools

import jax
import jax.numpy as jnp
import numpy as np
from jax import lax
from jax.experimental import pallas as pl
from jax.experimental.pallas import tpu as pltpu
from jax.experimental.pallas import tpu_sc as plsc

N = 10000
E = 320000
D = 128
ED = 16
H = 8
HD = 16
INV_SCALE = 1.0 / float(np.sqrt(HD))

NC = 2
NS = 16
NW = NC * NS
EPT = E // NW
CH = 16
NCHUNK = EPT // CH
NPAIR = (NCHUNK - 1) // 2
N_PAD = 10240
NP4 = N_PAD // 4
ACC_ROWS = N_PAD + NP4
TROWS = ACC_ROWS // NS


def _pre_body(x_ref, wqv_ref, bqv_ref, wk_ref, bk_ref, m_ref, qv_ref, kc_ref):
    xb = x_ref[...]
    qv_ref[...] = xb @ wqv_ref[...] + bqv_ref[...]
    k = xb @ wk_ref[...] + bk_ref[...]
    kc_ref[...] = jnp.concatenate([k, k @ m_ref[...]], axis=1)


def _tc_pre(x, wqv, bqv, wk, bk, m):
    bn = 1000
    grid = (N // bn,)
    return pl.pallas_call(
        _pre_body,
        grid=grid,
        in_specs=[
            pl.BlockSpec((bn, D), lambda i: (i, 0)),
            pl.BlockSpec((D, 2 * D), lambda i: (0, 0)),
            pl.BlockSpec((1, 2 * D), lambda i: (0, 0)),
            pl.BlockSpec((D, D), lambda i: (0, 0)),
            pl.BlockSpec((1, D), lambda i: (0, 0)),
            pl.BlockSpec((D, D), lambda i: (0, 0)),
        ],
        out_specs=[
            pl.BlockSpec((bn, 2 * D), lambda i: (i, 0)),
            pl.BlockSpec((bn, 2 * D), lambda i: (i, 0)),
        ],
        out_shape=[
            jax.ShapeDtypeStruct((N, 2 * D), jnp.float32),
            jax.ShapeDtypeStruct((N, 2 * D), jnp.float32),
        ],
    )(x, wqv, bqv, wk, bk, m)


def _sc_edge_kernel(qv_hbm, kc_hbm, idx_hbm, ea_hbm, z_hbm,
                    out_hbm,
                    ib0, ib1, qv0, qv1, kc0, kc1, ea0, ea1, wv, wp, acc,
                    semq0, semq1, semk0, semk1, seme0, seme1):
    cid = lax.axis_index("c")
    sub = lax.axis_index("s")
    wid = cid * NS + sub
    t0 = sub * TROWS

    pltpu.sync_copy(z_hbm, acc.at[pl.ds(t0, TROWS)])
    plsc.subcore_barrier()

    c0 = wid * NCHUNK
    iota16 = lax.iota(jnp.int32, 16)
    x8 = iota16 ^ 8
    x4 = iota16 ^ 4
    x2 = iota16 ^ 2
    x1 = iota16 ^ 1
    m8 = iota16 < 8
    m4 = (iota16 & 7) < 4
    m2 = (iota16 & 3) < 2

    def issue(cix, ib, qvb, kcb, eab, semq, semk, seme):
        pltpu.sync_copy(idx_hbm.at[cix], ib)
        pltpu.async_copy(ea_hbm.at[pl.ds(cix * CH, CH)], eab, seme)
        pltpu.async_copy(qv_hbm.at[ib.at[0]], qvb, semq)
        pltpu.async_copy(kc_hbm.at[ib.at[1]], kcb, semk)

    def drain(ib, qvb, kcb, eab, semq, semk, seme):
        pltpu.make_async_copy(qv_hbm.at[ib.at[0]], qvb, semq).wait()
        pltpu.make_async_copy(kc_hbm.at[ib.at[1]], kcb, semk).wait()
        pltpu.make_async_copy(ea_hbm.at[pl.ds(0, CH)], eab, seme).wait()

    def compute(ib, qvb, kcb, eab):
        dv = ib[1]
        for j in range(CH):
            d = dv[j]
            ea_v = eab[j, :]
            ps = []
            for h in range(H):
                q = qvb[j, pl.ds(h * HD, 16)]
                k = kcb[j, pl.ds(h * HD, 16)]
                c = kcb[j, pl.ds(D + h * HD, 16)]
                ps.append(q * k + ea_v * c)
            a = [p + p[x8] for p in ps]
            b = [jnp.where(m8, a[2 * i], a[2 * i + 1]) for i in range(4)]
            b = [t + t[x4] for t in b]
            cc = [jnp.where(m4, b[0], b[1]), jnp.where(m4, b[2], b[3])]
            cc = [t + t[x2] for t in cc]
            f = jnp.where(m2, cc[0], cc[1])
            f = f + f[x1]
            s = jnp.minimum(jnp.maximum(f * INV_SCALE, -5.0), 5.0)
            ef = jnp.exp(s)
            r = ef + ef[x8]
            r = r + r[x4]
            r = r + r[x2]
            r = r + r[x1]
            attn = r * (1.0 / 16.0)
            for h in range(H):
                v = qvb[j, pl.ds(D + h * HD, 16)]
                wv[j, pl.ds(h * HD, 16)] = v * attn
            ea_a = ea_v * attn
            at0 = jnp.where(iota16 == 0, attn, 0.0)
            for blk in range(4):
                sel = (d & 3) == blk
                wp[j, pl.ds(blk * 32, 16)] = jnp.where(sel, ea_a, 0.0)
                wp[j, pl.ds(blk * 32 + 16, 16)] = jnp.where(sel, at0, 0.0)
        pltpu.sync_copy(wv, acc.at[ib.at[1]], add=True)
        pltpu.sync_copy(wp, acc.at[ib.at[2]], add=True)

    issue(c0, ib0, qv0, kc0, ea0, semq0, semk0, seme0)

    def pair(g, carry):
        issue(c0 + 2 * g + 1, ib1, qv1, kc1, ea1, semq1, semk1, seme1)
        drain(ib0, qv0, kc0, ea0, semq0, semk0, seme0)
        compute(ib0, qv0, kc0, ea0)
        issue(c0 + 2 * g + 2, ib0, qv0, kc0, ea0, semq0, semk0, seme0)
        drain(ib1, qv1, kc1, ea1, semq1, semk1, seme1)
        compute(ib1, qv1, kc1, ea1)
        return carry

    lax.fori_loop(0, NPAIR, pair, 0)
    drain(ib0, qv0, kc0, ea0, semq0, semk0, seme0)
    compute(ib0, qv0, kc0, ea0)

    plsc.subcore_barrier()
    pltpu.sync_copy(acc.at[pl.ds(t0, TROWS)],
                    out_hbm.at[cid, pl.ds(t0, TROWS)])


def _sc_edge(qv, kc, idx3, ea, zeros):
    mesh = plsc.VectorSubcoreMesh(core_axis_name="c", subcore_axis_name="s")
    f = functools.partial(
        pl.kernel,
        out_type=jax.ShapeDtypeStruct((NC, ACC_ROWS, D), jnp.float32),
        mesh=mesh,
        scratch_types=[
            pltpu.VMEM((3, CH), jnp.int32),
            pltpu.VMEM((3, CH), jnp.int32),
            pltpu.VMEM((CH, 2 * D), jnp.float32),
            pltpu.VMEM((CH, 2 * D), jnp.float32),
            pltpu.VMEM((CH, 2 * D), jnp.float32),
            pltpu.VMEM((CH, 2 * D), jnp.float32),
            pltpu.VMEM((CH, ED), jnp.float32),
            pltpu.VMEM((CH, ED), jnp.float32),
            pltpu.VMEM((CH, D), jnp.float32),
            pltpu.VMEM((CH, D), jnp.float32),
            pltpu.VMEM_SHARED((ACC_ROWS, D), jnp.float32),
            pltpu.SemaphoreType.DMA,
            pltpu.SemaphoreType.DMA,
            pltpu.SemaphoreType.DMA,
            pltpu.SemaphoreType.DMA,
            pltpu.SemaphoreType.DMA,
            pltpu.SemaphoreType.DMA,
        ],
    )(_sc_edge_kernel)
    return f(qv, kc, idx3, ea, zeros)


def _post_body(v0_ref, v1_ref, s0_ref, s1_ref, x_ref, wev_ref, bev_ref,
               wo_ref, bo_ref, g_ref, b_ref, y_ref):
    a = v0_ref[...] + v1_ref[...]
    s = s0_ref[...] + s1_ref[...]
    bm = s[:, :ED]
    nrm = s[:, ED:ED + 1]
    acc = a + bm @ wev_ref[...] + nrm * bev_ref[...]
    out = acc / (nrm + 1e-8)
    out = out @ wo_ref[...] + bo_ref[...] + x_ref[...]
    mu = jnp.mean(out, axis=1, keepdims=True)
    var = jnp.mean((out - mu) ** 2, axis=1, keepdims=True)
    y_ref[...] = (out - mu) * lax.rsqrt(var + 1e-5) * g_ref[...] + b_ref[...]


def _tc_post(v0, v1, s0, s1, x, wev, bev, wo, bo, gamma, beta):
    bn = 1000
    grid = (N // bn,)
    return pl.pallas_call(
        _post_body,
        grid=grid,
        in_specs=[
            pl.BlockSpec((bn, D), lambda i: (i, 0)),
            pl.BlockSpec((bn, D), lambda i: (i, 0)),
            pl.BlockSpec((bn, 32), lambda i: (i, 0)),
            pl.BlockSpec((bn, 32), lambda i: (i, 0)),
            pl.BlockSpec((bn, D), lambda i: (i, 0)),
            pl.BlockSpec((ED, D), lambda i: (0, 0)),
            pl.BlockSpec((1, D), lambda i: (0, 0)),
            pl.BlockSpec((D, D), lambda i: (0, 0)),
            pl.BlockSpec((1, D), lambda i: (0, 0)),
            pl.BlockSpec((1, D), lambda i: (0, 0)),
            pl.BlockSpec((1, D), lambda i: (0, 0)),
        ],
        out_specs=pl.BlockSpec((bn, D), lambda i: (i, 0)),
        out_shape=jax.ShapeDtypeStruct((N, D), jnp.float32),
    )(v0, v1, s0, s1, x, wev, bev, wo, bo, gamma, beta)


def kernel(x, edge_index, edge_attr, Wq, bq, Wk, bk, Wv, bv, Wo, bo,
           Weq, beq, Wev, bev, gamma, beta):
    wqv = jnp.concatenate([Wq, Wv], axis=1)
    bqv = jnp.concatenate([bq + beq, bv])[None, :]
    blocks = [Weq[:, h * HD:(h + 1) * HD].T for h in range(H)]
    m = jax.scipy.linalg.block_diag(*blocks)

    src = edge_index[0]
    dst = edge_index[1]
    dst4 = jax.lax.shift_right_logical(dst, 2) + N_PAD
    idx3 = jnp.stack([src.reshape(-1, CH), dst.reshape(-1, CH),
                      dst4.reshape(-1, CH)], axis=1)
    zeros = jnp.zeros((TROWS, D), jnp.float32)

    qv, kc = _tc_pre(x, wqv, bqv, Wk, bk[None, :], m)
    acc = _sc_edge(qv, kc, idx3, edge_attr, zeros)
    acc_v = acc[:, :N_PAD]
    sp = acc[:, N_PAD:].reshape(NC, N_PAD, 32)
    y = _tc_post(acc_v[0, :N], acc_v[1, :N], sp[0, :N], sp[1, :N], x,
                 Wev, bev[None, :], Wo, bo[None, :], gamma[None, :],
                 beta[None, :])
    return y
